# baseline (speedup 1.0000x reference)
import functools

import jax
import jax.numpy as jnp
from jax import lax
from jax.experimental import pallas as pl
from jax.experimental.pallas import tpu as pltpu
from jax.experimental.pallas import tpu_sc as plsc

_LANE = 128
_J = 4
_GRP = _J * _LANE


@functools.partial(jax.jit, static_argnums=(2, 3, 4))
def _gather_call(xt3, pe, t_dim, d_dim, b_dim):
    info = plsc.get_sparse_core_info()
    nw = info.num_cores * info.num_subcores
    nb128 = b_dim // _LANE
    n_groups = t_dim * nb128 // _J
    gpw = n_groups // nw
    qpt = nb128 // _J
    d8 = d_dim // 8

    mesh = plsc.VectorSubcoreMesh(core_axis_name="c", subcore_axis_name="s")

    @functools.partial(
        pl.kernel,
        mesh=mesh,
        out_type=jax.ShapeDtypeStruct((t_dim, d8, nb128, 8, _LANE), jnp.float32),
        scratch_types=[
            pltpu.VMEM((2, _J, _LANE), jnp.int32),
            pltpu.VMEM((2, _GRP, d_dim), jnp.float32),
            pltpu.VMEM((2, d8, _J + 1, 8, _LANE + 5), jnp.float32),
            pltpu.SemaphoreType.DMA,
            pltpu.SemaphoreType.DMA,
            pltpu.SemaphoreType.DMA,
            pltpu.SemaphoreType.DMA,
            pltpu.SemaphoreType.DMA,
        ],
        compiler_params=pltpu.CompilerParams(
            use_tc_tiling_on_sc=False, needs_layout_passes=False
        ),
    )
    def k(xt_hbm, tab_hbm, out_hbm, idx_v, g_v, gt_v, gsem, is0, is1, ss0, ss1):
        isem = (is0, is1)
        ssem = (ss0, ss1)
        wid = lax.axis_index("s") * info.num_cores + lax.axis_index("c")
        g0 = wid * gpw

        def tq(i):
            gid = g0 + i
            return gid // qpt, gid % qpt

        def stage_idx(i, bb):
            t, q = tq(i)
            return pltpu.async_copy(
                xt_hbm.at[t, pl.ds(q * _J, _J)], idx_v.at[bb], isem[bb]
            )

        def fire_gathers(i, bb):
            return [
                pltpu.async_copy(
                    tab_hbm.at[idx_v.at[bb, j]],
                    g_v.at[bb, pl.ds(j * _LANE, _LANE)],
                    gsem,
                )
                for j in range(_J)
            ]

        def gt_slice(bb):
            return gt_v.at[bb, :, pl.ds(0, _J), :, pl.ds(0, _LANE)]

        def fire_store(i, bb):
            t, q = tq(i)
            return pltpu.async_copy(
                gt_slice(bb), out_hbm.at[t, :, pl.ds(q * _J, _J)], ssem[bb]
            )

        stage_idx(0, 0).wait()
        fire_gathers(0, 0)
        stage_idx(1, 1)

        iota = lax.iota(jnp.int32, 16)
        dt01 = iota // 8
        dt23 = dt01 + 2
        dr_vec = iota % 8

        def transpose_group(bb):
            for j in range(_J):
                jv = jnp.full((16,), j, jnp.int32)

                @plsc.parallel_loop(0, _LANE, unroll=16)
                def rbody(br):
                    row = g_v.at[bb, j * _LANE + br]
                    v0 = row[pl.ds(0, 16)]
                    v1 = row[pl.ds(16, 16)]
                    brv = jnp.full((16,), br, jnp.int32)
                    plsc.store_scatter(gt_v.at[bb], [dt01, jv, dr_vec, brv], v0)
                    plsc.store_scatter(gt_v.at[bb], [dt23, jv, dr_vec, brv], v1)

        def body(i2, carry):
            for bb in range(2):
                i = i2 * 2 + bb
                b1 = 1 - bb
                for j in range(_J):
                    pltpu.make_async_copy(
                        tab_hbm.at[idx_v.at[bb, j]],
                        g_v.at[bb, pl.ds(j * _LANE, _LANE)],
                        gsem,
                    ).wait()

                @pl.when(i + 1 < gpw)
                def _():
                    pltpu.make_async_copy(
                        xt_hbm.at[0, pl.ds(0, _J)], idx_v.at[b1], isem[b1]
                    ).wait()
                    fire_gathers(i + 1, b1)

                @pl.when(i + 2 < gpw)
                def _():
                    stage_idx(i + 2, bb)

                @pl.when(i >= 2)
                def _():
                    pltpu.make_async_copy(
                        gt_slice(bb), out_hbm.at[0, :, pl.ds(0, _J)], ssem[bb]
                    ).wait()

                transpose_group(bb)
                fire_store(i, bb)
            return carry

        lax.fori_loop(0, gpw // 2, body, 0)

        for bb in range(2):
            pltpu.make_async_copy(
                gt_slice(bb), out_hbm.at[0, :, pl.ds(0, _J)], ssem[bb]
            ).wait()

    return k(xt3, pe)


def kernel(x, pe):
    b, t = x.shape
    v, d = pe.shape
    xt3 = x.T.astype(jnp.int32).reshape(t, b // _LANE, _LANE)
    po = _gather_call(xt3, pe, t, d, b)
    a = jnp.transpose(po, (0, 1, 3, 2, 4))
    bb = a.reshape(t, d, b)
    return jnp.transpose(bb, (2, 0, 1))


# speedup vs baseline: 1.0313x; 1.0313x over previous
import functools

import jax
import jax.numpy as jnp
from jax import lax
from jax.experimental import pallas as pl
from jax.experimental.pallas import tpu as pltpu
from jax.experimental.pallas import tpu_sc as plsc

_LANE = 128
_J = 4
_GRP = _J * _LANE


@functools.partial(jax.jit, static_argnums=(2, 3, 4))
def _gather_call(xt3, pe, t_dim, d_dim, b_dim):
    info = plsc.get_sparse_core_info()
    nw = info.num_cores * info.num_subcores
    nb128 = b_dim // _LANE
    n_groups = t_dim * nb128 // _J
    gpw = n_groups // nw
    qpt = nb128 // _J
    d8 = d_dim // 8

    mesh = plsc.VectorSubcoreMesh(core_axis_name="c", subcore_axis_name="s")

    @functools.partial(
        pl.kernel,
        mesh=mesh,
        out_type=jax.ShapeDtypeStruct((t_dim, d8, nb128, 8, _LANE), jnp.float32),
        scratch_types=[
            pltpu.VMEM((2, _J, _LANE), jnp.int32),
            pltpu.VMEM((2, _GRP, d_dim), jnp.float32),
            pltpu.VMEM((2, d8, _J + 1, 8, _LANE + 5), jnp.float32),
            pltpu.SemaphoreType.DMA,
            pltpu.SemaphoreType.DMA,
            pltpu.SemaphoreType.DMA,
            pltpu.SemaphoreType.DMA,
            pltpu.SemaphoreType.DMA,
            pltpu.SemaphoreType.DMA,
        ],
        compiler_params=pltpu.CompilerParams(
            use_tc_tiling_on_sc=False, needs_layout_passes=False
        ),
    )
    def k(xt_hbm, tab_hbm, out_hbm, idx_v, g_v, gt_v, gs0, gs1, is0, is1, ss0, ss1):
        gsem = (gs0, gs1)
        isem = (is0, is1)
        ssem = (ss0, ss1)
        wid = lax.axis_index("s") * info.num_cores + lax.axis_index("c")
        g0 = wid * gpw

        def tq(i):
            gid = g0 + i
            return gid // qpt, gid % qpt

        def stage_idx(i, bb):
            t, q = tq(i)
            return pltpu.async_copy(
                xt_hbm.at[t, pl.ds(q * _J, _J)], idx_v.at[bb], isem[bb]
            )

        def fire_gathers(i, bb):
            return [
                pltpu.async_copy(
                    tab_hbm.at[idx_v.at[bb, j]],
                    g_v.at[bb, pl.ds(j * _LANE, _LANE)],
                    gsem[bb],
                )
                for j in range(_J)
            ]

        def gt_slice(bb):
            return gt_v.at[bb, :, pl.ds(0, _J), :, pl.ds(0, _LANE)]

        def fire_store(i, bb):
            t, q = tq(i)
            return pltpu.async_copy(
                gt_slice(bb), out_hbm.at[t, :, pl.ds(q * _J, _J)], ssem[bb]
            )

        stage_idx(0, 0).wait()
        fire_gathers(0, 0)
        stage_idx(1, 1)

        iota = lax.iota(jnp.int32, 16)
        dt01 = iota // 8
        dt23 = dt01 + 2
        dr_vec = iota % 8

        def transpose_group(bb):
            for j in range(_J):
                jv = jnp.full((16,), j, jnp.int32)

                @plsc.parallel_loop(0, _LANE, unroll=8)
                def rbody(br):
                    row = g_v.at[bb, j * _LANE + br]
                    v0 = row[pl.ds(0, 16)]
                    v1 = row[pl.ds(16, 16)]
                    brv = jnp.full((16,), br, jnp.int32)
                    plsc.store_scatter(gt_v.at[bb], [dt01, jv, dr_vec, brv], v0)
                    plsc.store_scatter(gt_v.at[bb], [dt23, jv, dr_vec, brv], v1)

        def body(i2, carry):
            for bb in range(2):
                i = i2 * 2 + bb
                b1 = 1 - bb

                @pl.when(i + 1 < gpw)
                def _():
                    pltpu.make_async_copy(
                        xt_hbm.at[0, pl.ds(0, _J)], idx_v.at[b1], isem[b1]
                    ).wait()
                    fire_gathers(i + 1, b1)

                for j in range(_J):
                    pltpu.make_async_copy(
                        tab_hbm.at[idx_v.at[bb, j]],
                        g_v.at[bb, pl.ds(j * _LANE, _LANE)],
                        gsem[bb],
                    ).wait()

                @pl.when(i + 2 < gpw)
                def _():
                    stage_idx(i + 2, bb)

                @pl.when(i >= 2)
                def _():
                    pltpu.make_async_copy(
                        gt_slice(bb), out_hbm.at[0, :, pl.ds(0, _J)], ssem[bb]
                    ).wait()

                transpose_group(bb)
                fire_store(i, bb)
            return carry

        lax.fori_loop(0, gpw // 2, body, 0)

        for bb in range(2):
            pltpu.make_async_copy(
                gt_slice(bb), out_hbm.at[0, :, pl.ds(0, _J)], ssem[bb]
            ).wait()

    return k(xt3, pe)


def kernel(x, pe):
    b, t = x.shape
    v, d = pe.shape
    xt3 = x.T.astype(jnp.int32).reshape(t, b // _LANE, _LANE)
    po = _gather_call(xt3, pe, t, d, b)
    a = jnp.transpose(po, (0, 1, 3, 2, 4))
    bb = a.reshape(t, d, b)
    return jnp.transpose(bb, (2, 0, 1))


# speedup vs baseline: 1.0620x; 1.0297x over previous
import functools

import jax
import jax.numpy as jnp
from jax import lax
from jax.experimental import pallas as pl
from jax.experimental.pallas import tpu as pltpu
from jax.experimental.pallas import tpu_sc as plsc

_LANE = 128
_J = 4
_GRP = _J * _LANE


@functools.partial(jax.jit, static_argnums=(2, 3, 4))
def _gather_call(xt3, pe, t_dim, d_dim, b_dim):
    info = plsc.get_sparse_core_info()
    nw = info.num_cores * info.num_subcores
    nb128 = b_dim // _LANE
    n_groups = t_dim * nb128 // _J
    gpw = n_groups // nw
    qpt = nb128 // _J
    d8 = d_dim // 8

    mesh = plsc.VectorSubcoreMesh(core_axis_name="c", subcore_axis_name="s")

    @functools.partial(
        pl.kernel,
        mesh=mesh,
        out_type=jax.ShapeDtypeStruct((t_dim, d8, nb128, 8, _LANE), jnp.float32),
        scratch_types=[
            pltpu.VMEM((2, _J, _LANE), jnp.int32),
            pltpu.VMEM((2, _GRP, d_dim), jnp.float32),
            pltpu.VMEM((2, d8, _J + 1, 8, _LANE + 5), jnp.float32),
            pltpu.SemaphoreType.DMA,
            pltpu.SemaphoreType.DMA,
            pltpu.SemaphoreType.DMA,
            pltpu.SemaphoreType.DMA,
            pltpu.SemaphoreType.DMA,
            pltpu.SemaphoreType.DMA,
        ],
        compiler_params=pltpu.CompilerParams(
            use_tc_tiling_on_sc=False, needs_layout_passes=False
        ),
    )
    def k(xt_hbm, tab_hbm, out_hbm, idx_v, g_v, gt_v, gs0, gs1, is0, is1, ss0, ss1):
        gsem = (gs0, gs1)
        isem = (is0, is1)
        ssem = (ss0, ss1)
        wid = lax.axis_index("s") * info.num_cores + lax.axis_index("c")
        g0 = wid * gpw

        def tq(i):
            gid = g0 + i
            return gid // qpt, gid % qpt

        def stage_idx(i, bb):
            t, q = tq(i)
            return pltpu.async_copy(
                xt_hbm.at[t, pl.ds(q * _J, _J)], idx_v.at[bb], isem[bb]
            )

        def fire_gathers(i, bb):
            return [
                pltpu.async_copy(
                    tab_hbm.at[idx_v.at[bb, j]],
                    g_v.at[bb, pl.ds(j * _LANE, _LANE)],
                    gsem[bb],
                )
                for j in range(_J)
            ]

        def gt_slice(bb):
            return gt_v.at[bb, :, pl.ds(0, _J), :, pl.ds(0, _LANE)]

        def fire_store(i, bb):
            t, q = tq(i)
            return pltpu.async_copy(
                gt_slice(bb), out_hbm.at[t, :, pl.ds(q * _J, _J)], ssem[bb]
            )

        stage_idx(0, 0).wait()
        fire_gathers(0, 0)
        stage_idx(1, 1)

        iota = lax.iota(jnp.int32, 16)
        dt01 = iota // 8
        dt23 = dt01 + 2
        dr_vec = iota % 8

        def transpose_group(bb):
            for j in range(_J):
                jv = jnp.full((16,), j, jnp.int32)

                @plsc.parallel_loop(0, _LANE, unroll=8)
                def rbody(br):
                    row = g_v.at[bb, j * _LANE + br]
                    v0 = row[pl.ds(0, 16)]
                    v1 = row[pl.ds(16, 16)]
                    brv = jnp.full((16,), br, jnp.int32)
                    plsc.store_scatter(gt_v.at[bb], [dt01, jv, dr_vec, brv], v0)
                    plsc.store_scatter(gt_v.at[bb], [dt23, jv, dr_vec, brv], v1)

        def body(i2, carry):
            for bb in range(2):
                i = i2 * 2 + bb
                b1 = 1 - bb

                @pl.when(i + 1 < gpw)
                def _():
                    pltpu.make_async_copy(
                        xt_hbm.at[0, pl.ds(0, _J)], idx_v.at[b1], isem[b1]
                    ).wait()
                    fire_gathers(i + 1, b1)

                for j in range(_J):
                    pltpu.make_async_copy(
                        tab_hbm.at[idx_v.at[bb, j]],
                        g_v.at[bb, pl.ds(j * _LANE, _LANE)],
                        gsem[bb],
                    ).wait()

                @pl.when(i + 2 < gpw)
                def _():
                    stage_idx(i + 2, bb)

                @pl.when(i >= 2)
                def _():
                    pltpu.make_async_copy(
                        gt_slice(bb), out_hbm.at[0, :, pl.ds(0, _J)], ssem[bb]
                    ).wait()

                transpose_group(bb)
                fire_store(i, bb)
            return carry

        lax.fori_loop(0, gpw // 2, body, 0)

        for bb in range(2):
            pltpu.make_async_copy(
                gt_slice(bb), out_hbm.at[0, :, pl.ds(0, _J)], ssem[bb]
            ).wait()

    return k(xt3, pe)


def kernel(x, pe):
    b, t = x.shape
    v, d = pe.shape
    tab4 = jnp.pad(pe, ((0, 0), (0, 128 - d))).reshape(v * (128 // d), d)
    xt3 = (x.T.astype(jnp.int32) * (128 // d)).reshape(t, b // _LANE, _LANE)
    po = _gather_call(xt3, tab4, t, d, b)
    a = jnp.transpose(po, (0, 1, 3, 2, 4))
    bb = a.reshape(t, d, b)
    return jnp.transpose(bb, (2, 0, 1))
